# Initial kernel scaffold; baseline (speedup 1.0000x reference)
import jax
import jax.numpy as jnp
from jax.experimental import pallas as pl


def kernel(frames, x_obj, x_room, x_attr, W_int, b_int, W_beh, b_beh, W2, b2, src_oo, dst_oo, src_ro, dst_ro, src_ao, dst_ao):
    raise NotImplementedError("write your pallas kernel here")


# speedup vs baseline: 2.4000x; 2.4000x over previous
import functools

import jax
import jax.numpy as jnp
from jax import lax
from jax.experimental import pallas as pl
from jax.experimental.pallas import tpu as pltpu
from jax.experimental.pallas import tpu_sc as plsc

N_OBJ, N_ROOM, N_ATTR = 10000, 100, 500
E_OO, E_RO, E_AO = 320000, 50000, 50000
D = 128

NW = 32
NROW = 10240
JUNK_ROW = 10200

OFF_SRC_OO = 0
OFF_DST_OO = 10240
OFF_DST_RO = 20480
OFF_DST_AO = 30720
OFF_SRC_RO = 40960
OFF_SRC_AO = 41088
CNT_TOTAL = 49152
JUNK_CNT = 41600

CNT_CHUNK = 2688
CNT_NCHUNK = 10
CNT_PER_W = CNT_CHUNK * CNT_NCHUNK
CNT_IDX_TOTAL = NW * CNT_PER_W

G_OO = 80
G_RO = 13

_mesh = plsc.VectorSubcoreMesh(core_axis_name="c", subcore_axis_name="s")
_sc_params = pltpu.CompilerParams(needs_layout_passes=False)


@functools.partial(
    pl.kernel,
    out_type=jax.ShapeDtypeStruct((NW, CNT_TOTAL), jnp.float32),
    mesh=_mesh,
    compiler_params=_sc_params,
    scratch_types=[
        pltpu.VMEM((CNT_TOTAL,), jnp.float32),
        pltpu.VMEM((CNT_CHUNK,), jnp.int32),
    ],
)
def _sc_counts(idx_hbm, out_hbm, cnt, ibuf):
    c = lax.axis_index("c")
    s = lax.axis_index("s")
    w = c * 16 + s
    z16 = jnp.zeros((16,), jnp.float32)
    ones = jnp.ones((16,), jnp.float32)

    def zero_body(i, _):
        cnt[pl.ds(i * 16, 16)] = z16
        return 0

    lax.fori_loop(0, CNT_TOTAL // 16, zero_body, 0)

    def chunk_body(ci, _):
        pltpu.sync_copy(idx_hbm.at[pl.ds(w * CNT_PER_W + ci * CNT_CHUNK, CNT_CHUNK)], ibuf)

        def inner(i, _):
            iv = ibuf[pl.ds(i * 16, 16)]
            plsc.addupdate_scatter(cnt, [iv], ones)
            return 0

        lax.fori_loop(0, CNT_CHUNK // 16, inner, 0)
        return 0

    lax.fori_loop(0, CNT_NCHUNK, chunk_body, 0)
    pltpu.sync_copy(cnt, out_hbm.at[w])


def _make_sc_agg(phases):
    n_ph = len(phases)
    out_type = tuple(
        jax.ShapeDtypeStruct((2, NROW, D), jnp.float32) for _ in range(n_ph)
    )

    @functools.partial(
        pl.kernel,
        out_type=out_type,
        mesh=_mesh,
        compiler_params=_sc_params,
        scratch_types=[
            pltpu.VMEM_SHARED((NROW, D), jnp.float32),
            pltpu.VMEM((128, D), jnp.float32),
            pltpu.VMEM((16, 128), jnp.int32),
            pltpu.VMEM((16, 128), jnp.int32),
            pltpu.SemaphoreType.DMA,
        ],
    )
    def agg(*refs):
        ins = refs[: 3 * n_ph]
        outs = refs[3 * n_ph : 4 * n_ph]
        acc, rbuf, sbuf, dbuf, gsem = refs[4 * n_ph :]
        c = lax.axis_index("c")
        s = lax.axis_index("s")
        w = c * 16 + s
        z16 = jnp.zeros((16,), jnp.float32)
        stripe = NROW // 16

        for ph in range(n_ph):
            p_hbm, sidx_hbm, didx_hbm = ins[3 * ph : 3 * ph + 3]
            out_hbm = outs[ph]
            G = phases[ph][1]

            def zb(i, _):
                rbuf[i // 8, pl.ds((i % 8) * 16, 16)] = z16
                return 0

            lax.fori_loop(0, 128 * 8, zb, 0)
            for k in range(stripe // 128):
                pltpu.sync_copy(rbuf, acc.at[pl.ds(s * stripe + k * 128, 128)])
            plsc.subcore_barrier()

            nchunks = (G + 15) // 16
            for ch in range(nchunks):
                m = min(16, G - ch * 16)
                g0 = ch * 16
                pltpu.sync_copy(sidx_hbm.at[w, pl.ds(g0, m)], sbuf.at[pl.ds(0, m)])
                pltpu.sync_copy(didx_hbm.at[w, pl.ds(g0, m)], dbuf.at[pl.ds(0, m)])

                def gbody(j, _):
                    pltpu.async_copy(p_hbm.at[sbuf.at[j]], rbuf, gsem).wait()
                    pltpu.sync_copy(rbuf, acc.at[dbuf.at[j]], add=True)
                    return 0

                lax.fori_loop(0, m, gbody, 0)
            plsc.subcore_barrier()

            for k in range(stripe // 128):
                r0 = s * stripe + k * 128
                pltpu.sync_copy(acc.at[pl.ds(r0, 128)], out_hbm.at[c, pl.ds(r0, 128)])
            plsc.subcore_barrier()

    return agg


_sc_agg3 = _make_sc_agg([(N_OBJ, G_OO), (N_ROOM, G_RO), (N_ATTR, G_RO)])
_sc_agg1 = _make_sc_agg([(NROW, G_OO)])


def _tc_scales(cnt_parts):
    def body(c_ref, o_ref):
        tot = jnp.sum(c_ref[...], axis=0)
        o_ref[...] = lax.rsqrt(jnp.maximum(tot, 1.0))

    return pl.pallas_call(
        body,
        grid=(3,),
        in_specs=[pl.BlockSpec((NW, 128, 128), lambda i: (0, i, 0))],
        out_specs=pl.BlockSpec((128, 128), lambda i: (i, 0)),
        out_shape=jax.ShapeDtypeStruct((384, 128), jnp.float32),
    )(cnt_parts)


def _tc_proj(x, scol, W, blk):
    n = x.shape[0]

    def body(x_ref, s_ref, w_ref, o_ref):
        o_ref[...] = jnp.dot(
            x_ref[...] * s_ref[...], w_ref[...], preferred_element_type=jnp.float32
        )

    return pl.pallas_call(
        body,
        grid=(n // blk,),
        in_specs=[
            pl.BlockSpec((blk, D), lambda i: (i, 0)),
            pl.BlockSpec((blk, 1), lambda i: (i, 0)),
            pl.BlockSpec((D, D), lambda i: (0, 0)),
        ],
        out_specs=pl.BlockSpec((blk, D), lambda i: (i, 0)),
        out_shape=jax.ShapeDtypeStruct((n, D), jnp.float32),
    )(x, scol, W)


def _tc_combine(o1, o2, o3, s1, s2, s3, s4, b_int, b_beh, W2):
    blk = 1024

    def body(o1r, o2r, o3r, s1r, s2r, s3r, s4r, bir, bbr, wr, qr):
        a1 = (o1r[0] + o1r[1]) * s1r[...] + bir[...]
        a2 = (o2r[0] + o2r[1]) * s2r[...] + bir[...]
        a3 = (o3r[0] + o3r[1]) * s3r[...] + bbr[...]
        h = (jnp.maximum(a1, 0.0) + jnp.maximum(a2, 0.0) + jnp.maximum(a3, 0.0)) * (
            1.0 / 3.0
        )
        qr[...] = jnp.dot(h * s4r[...], wr[...], preferred_element_type=jnp.float32)

    part = pl.BlockSpec((2, blk, D), lambda i: (0, i, 0))
    scol = pl.BlockSpec((blk, 1), lambda i: (i, 0))
    brow = pl.BlockSpec((1, D), lambda i: (0, 0))
    return pl.pallas_call(
        body,
        grid=(NROW // blk,),
        in_specs=[part, part, part, scol, scol, scol, scol, brow, brow,
                  pl.BlockSpec((D, D), lambda i: (0, 0))],
        out_specs=pl.BlockSpec((blk, D), lambda i: (i, 0)),
        out_shape=jax.ShapeDtypeStruct((NROW, D), jnp.float32),
    )(o1, o2, o3, s1, s2, s3, s4, b_int, b_beh, W2)


def _tc_final(o4, scol, b2, nb):
    blk = 1000

    def body(o_ref, s_ref, b_ref, out_ref):
        h2 = (o_ref[0] + o_ref[1]) * s_ref[...] + b_ref[...]
        out_ref[...] = h2[None]

    return pl.pallas_call(
        body,
        grid=(nb, N_OBJ // blk),
        in_specs=[
            pl.BlockSpec((2, blk, D), lambda b, i: (0, i, 0)),
            pl.BlockSpec((blk, 1), lambda b, i: (i, 0)),
            pl.BlockSpec((1, D), lambda b, i: (0, 0)),
        ],
        out_specs=pl.BlockSpec((1, blk, D), lambda b, i: (b, i, 0)),
        out_shape=jax.ShapeDtypeStruct((nb, N_OBJ, D), jnp.float32),
    )(o4, scol, b2)


def _pad_edges(src, dst, total):
    ns = total - src.shape[0]
    ps = jnp.concatenate([src, jnp.zeros((ns,), jnp.int32)])
    pd = jnp.concatenate([dst, jnp.full((ns,), JUNK_ROW, jnp.int32)])
    return ps.reshape(NW, -1, 128), pd.reshape(NW, -1, 128)


def kernel(frames, x_obj, x_room, x_attr, W_int, b_int, W_beh, b_beh, W2, b2,
           src_oo, dst_oo, src_ro, dst_ro, src_ao, dst_ao):
    nb = frames.shape[0]

    idx_all = jnp.concatenate([
        src_oo + OFF_SRC_OO, dst_oo + OFF_DST_OO,
        dst_ro + OFF_DST_RO, dst_ao + OFF_DST_AO,
        src_ro + OFF_SRC_RO, src_ao + OFF_SRC_AO,
        jnp.full((CNT_IDX_TOTAL - (2 * E_OO + 4 * E_RO),), JUNK_CNT, jnp.int32),
    ])
    soo, doo = _pad_edges(src_oo, dst_oo, NW * G_OO * 128)
    sro, dro = _pad_edges(src_ro, dst_ro, NW * G_RO * 128)
    sao, dao = _pad_edges(src_ao, dst_ao, NW * G_RO * 128)

    cnt_parts = _sc_counts(idx_all).reshape(NW, 384, 128)
    flat = _tc_scales(cnt_parts).reshape(-1)
    s_src_oo = flat[OFF_SRC_OO:OFF_SRC_OO + NROW, None]
    s_dst_oo = flat[OFF_DST_OO:OFF_DST_OO + NROW, None]
    s_dst_ro = flat[OFF_DST_RO:OFF_DST_RO + NROW, None]
    s_dst_ao = flat[OFF_DST_AO:OFF_DST_AO + NROW, None]
    s_src_ro = flat[OFF_SRC_RO:OFF_SRC_RO + N_ROOM, None]
    s_src_ao = flat[OFF_SRC_AO:OFF_SRC_AO + N_ATTR, None]

    p1 = _tc_proj(x_obj, s_src_oo[:N_OBJ], W_int, 1000)
    p2 = _tc_proj(x_room, s_src_ro, W_int, N_ROOM)
    p3 = _tc_proj(x_attr, s_src_ao, W_beh, N_ATTR)
    o1, o2, o3 = _sc_agg3(p1, soo, doo, p2, sro, dro, p3, sao, dao)

    q = _tc_combine(o1, o2, o3, s_dst_oo, s_dst_ro, s_dst_ao, s_src_oo,
                    b_int.reshape(1, D), b_beh.reshape(1, D), W2)
    (o4,) = _sc_agg1(q, soo, doo)
    return _tc_final(o4, s_dst_oo[:N_OBJ], b2.reshape(1, D), nb)
